# Initial kernel scaffold; baseline (speedup 1.0000x reference)
import jax
import jax.numpy as jnp
from jax.experimental import pallas as pl


def kernel(x, edge_index, batch, W1, b1, W2, b2, Wfc, bfc):
    raise NotImplementedError("write your pallas kernel here")


# speedup vs baseline: 10.5903x; 10.5903x over previous
import functools

import jax
import jax.numpy as jnp
from jax import lax
from jax.experimental import pallas as pl
from jax.experimental.pallas import tpu as pltpu
from jax.experimental.pallas import tpu_sc as plsc

NC = 2
NS = 16
NW = NC * NS
K = 128
BLK = 256

F32 = jnp.float32


def _deg_body(dst_hbm, out_hbm, didx, ones_v, zb, acc, *, ept, npad):
  cid = lax.axis_index("c")
  sid = lax.axis_index("s")
  wid = cid * NS + sid
  rpt = npad // NS
  for c in range(8):
    zb[pl.ds(c * 16, 16)] = jnp.zeros((16,), F32)
    ones_v[pl.ds(c * 16, 16)] = jnp.full((16,), 1.0, F32)
  r0 = sid * rpt
  for t in range(rpt // K):
    pltpu.sync_copy(zb, acc.at[pl.ds(r0 + t * K, K)])
  plsc.subcore_barrier()
  base = wid * ept

  def body(j, carry):
    pltpu.sync_copy(dst_hbm.at[pl.ds(base + j * K, K)], didx)
    pltpu.sync_copy(ones_v, acc.at[didx], add=True)
    return carry

  lax.fori_loop(0, ept // K, body, 0)
  plsc.subcore_barrier()
  for t in range(rpt // K):
    pltpu.sync_copy(acc.at[pl.ds(r0 + t * K, K)],
                    out_hbm.at[cid, pl.ds(r0 + t * K, K)])


def _agg_body(zs_hbm, src_hbm, dst_hbm, out_hbm, sidx, didx, rows, zb, acc,
              *, ept, npad):
  cid = lax.axis_index("c")
  sid = lax.axis_index("s")
  wid = cid * NS + sid
  rpt = npad // NS
  for i in range(16):
    for c in range(8):
      zb[i, pl.ds(c * 16, 16)] = jnp.zeros((16,), F32)
  r0 = sid * rpt
  for t in range(rpt // 16):
    pltpu.sync_copy(zb, acc.at[pl.ds(r0 + t * 16, 16)])
  plsc.subcore_barrier()
  base = wid * ept

  def body(j, carry):
    off = base + j * K
    pltpu.sync_copy(src_hbm.at[pl.ds(off, K)], sidx)
    pltpu.sync_copy(dst_hbm.at[pl.ds(off, K)], didx)
    pltpu.sync_copy(zs_hbm.at[sidx], rows)
    pltpu.sync_copy(rows, acc.at[didx], add=True)
    return carry

  lax.fori_loop(0, ept // K, body, 0)
  plsc.subcore_barrier()
  for t in range(rpt // K):
    pltpu.sync_copy(acc.at[pl.ds(r0 + t * K, K)],
                    out_hbm.at[cid, pl.ds(r0 + t * K, K)])


def _sc_deg(dst_pad, ept, npad):
  mesh = plsc.VectorSubcoreMesh(core_axis_name="c", subcore_axis_name="s")
  fn = pl.kernel(
      functools.partial(_deg_body, ept=ept, npad=npad),
      mesh=mesh,
      out_type=jax.ShapeDtypeStruct((NC, npad), F32),
      scratch_types=[
          pltpu.VMEM((K,), jnp.int32),
          pltpu.VMEM((K,), F32),
          pltpu.VMEM((K,), F32),
          pltpu.VMEM_SHARED((npad,), F32),
      ],
  )
  return fn(dst_pad)


def _sc_agg(zs, src_pad, dst_pad, ept, npad):
  mesh = plsc.VectorSubcoreMesh(core_axis_name="c", subcore_axis_name="s")
  fn = pl.kernel(
      functools.partial(_agg_body, ept=ept, npad=npad),
      mesh=mesh,
      out_type=jax.ShapeDtypeStruct((NC, npad, 128), F32),
      scratch_types=[
          pltpu.VMEM((K,), jnp.int32),
          pltpu.VMEM((K,), jnp.int32),
          pltpu.VMEM((K, 128), F32),
          pltpu.VMEM((16, 128), F32),
          pltpu.VMEM_SHARED((npad, 128), F32),
      ],
  )
  return fn(zs, src_pad, dst_pad)


def _dinv_blk(degp_ref, i):
  d = degp_ref[:, pl.ds(i * BLK, BLK)]
  return lax.rsqrt(d[0] + d[1] + 1.0)


def _mm_scale_body(x_ref, w_ref, degp_ref, o_ref):
  i = pl.program_id(0)
  dinv = _dinv_blk(degp_ref, i)
  z = jnp.dot(x_ref[...], w_ref[...], preferred_element_type=F32)
  o_ref[...] = z * dinv[:, None]


def _mid_body(agg_ref, zs_ref, degp_ref, b_ref, w_ref, o_ref):
  i = pl.program_id(0)
  dinv = _dinv_blk(degp_ref, i)[:, None]
  h = (agg_ref[0] + agg_ref[1] + zs_ref[...]) * dinv + b_ref[...]
  h = jnp.maximum(h, 0.0)
  o_ref[...] = jnp.dot(h, w_ref[...], preferred_element_type=F32) * dinv


def _pool_body(agg_ref, zs_ref, degp_ref, b_ref, batch_ref, wfc_ref, bfc_ref,
               o_ref, pacc, cacc, *, nblk, g):
  i = pl.program_id(0)
  dinv = _dinv_blk(degp_ref, i)[:, None]
  h = (agg_ref[0] + agg_ref[1] + zs_ref[...]) * dinv + b_ref[...]
  h = jnp.maximum(h, 0.0)
  bt = batch_ref[0, 0, :]
  mask = (bt[:, None] == lax.broadcasted_iota(jnp.int32, (BLK, g), 1))
  mask = mask.astype(F32)

  @pl.when(i == 0)
  def _():
    pacc[...] = jnp.zeros_like(pacc)
    cacc[...] = jnp.zeros_like(cacc)

  pacc[...] += lax.dot_general(mask, h, (((0,), (0,)), ((), ())),
                               preferred_element_type=F32)
  cacc[...] += jnp.sum(mask, axis=0)[None, :]

  @pl.when(i == nblk - 1)
  def _():
    cnt = jnp.maximum(cacc[...], 1.0).reshape(g, 1)
    pooled = pacc[...] / cnt
    o_ref[...] = (jnp.dot(pooled, wfc_ref[...], preferred_element_type=F32)
                  + bfc_ref[...])


def _tc_mm_scale(x_pad, w, degp, npad):
  nblk = npad // BLK
  return pl.pallas_call(
      _mm_scale_body,
      grid=(nblk,),
      in_specs=[
          pl.BlockSpec((BLK, 128), lambda i: (i, 0)),
          pl.BlockSpec((128, 128), lambda i: (0, 0)),
          pl.BlockSpec((NC, npad), lambda i: (0, 0)),
      ],
      out_specs=pl.BlockSpec((BLK, 128), lambda i: (i, 0)),
      out_shape=jax.ShapeDtypeStruct((npad, 128), F32),
  )(x_pad, w, degp)


def _tc_mid(agg, zs, degp, b, w, npad):
  nblk = npad // BLK
  return pl.pallas_call(
      _mid_body,
      grid=(nblk,),
      in_specs=[
          pl.BlockSpec((NC, BLK, 128), lambda i: (0, i, 0)),
          pl.BlockSpec((BLK, 128), lambda i: (i, 0)),
          pl.BlockSpec((NC, npad), lambda i: (0, 0)),
          pl.BlockSpec((1, 128), lambda i: (0, 0)),
          pl.BlockSpec((128, 128), lambda i: (0, 0)),
      ],
      out_specs=pl.BlockSpec((BLK, 128), lambda i: (i, 0)),
      out_shape=jax.ShapeDtypeStruct((npad, 128), F32),
  )(agg, zs, degp, b.reshape(1, 128), w)


def _tc_pool(agg, zs, degp, b, batch3, wfc, bfc, npad, g, c):
  nblk = npad // BLK
  return pl.pallas_call(
      functools.partial(_pool_body, nblk=nblk, g=g),
      grid=(nblk,),
      in_specs=[
          pl.BlockSpec((NC, BLK, 128), lambda i: (0, i, 0)),
          pl.BlockSpec((BLK, 128), lambda i: (i, 0)),
          pl.BlockSpec((NC, npad), lambda i: (0, 0)),
          pl.BlockSpec((1, 128), lambda i: (0, 0)),
          pl.BlockSpec((1, 1, BLK), lambda i: (i, 0, 0)),
          pl.BlockSpec((128, c), lambda i: (0, 0)),
          pl.BlockSpec((1, c), lambda i: (0, 0)),
      ],
      out_specs=pl.BlockSpec((g, c), lambda i: (0, 0)),
      out_shape=jax.ShapeDtypeStruct((g, c), F32),
      scratch_shapes=[
          pltpu.VMEM((g, 128), F32),
          pltpu.VMEM((1, g), F32),
      ],
  )(agg, zs, degp, b.reshape(1, 128), batch3, wfc, bfc.reshape(1, c))


def kernel(x, edge_index, batch, W1, b1, W2, b2, Wfc, bfc):
  n, d = x.shape
  e = edge_index.shape[1]
  g = 64
  c = Wfc.shape[1]

  npad = ((n + BLK) // BLK) * BLK
  ept = -(-e // (NW * K)) * K
  epad = ept * NW

  x_pad = jnp.pad(x, ((0, npad - n), (0, 0)))
  src_pad = jnp.concatenate(
      [edge_index[0], jnp.zeros((epad - e,), jnp.int32)])
  dst_pad = jnp.concatenate(
      [edge_index[1], jnp.full((epad - e,), n, jnp.int32)])
  batch3 = jnp.concatenate(
      [batch, jnp.full((npad - n,), -1, jnp.int32)]).reshape(-1, 1, BLK)

  degp = _sc_deg(dst_pad, ept, npad)
  zs1 = _tc_mm_scale(x_pad, W1, degp, npad)
  agg1 = _sc_agg(zs1, src_pad, dst_pad, ept, npad)
  zs2 = _tc_mid(agg1, zs1, degp, b1, W2, npad)
  agg2 = _sc_agg(zs2, src_pad, dst_pad, ept, npad)
  return _tc_pool(agg2, zs2, degp, b2, batch3, Wfc, bfc, npad, g, c)
